# Initial kernel scaffold; baseline (speedup 1.0000x reference)
import jax
import jax.numpy as jnp
from jax.experimental import pallas as pl


def kernel(x, edge_idx, edge_wt, W1, b1, W2, b2):
    raise NotImplementedError("write your pallas kernel here")


# speedup vs baseline: 6.5579x; 6.5579x over previous
import functools
import jax
import jax.numpy as jnp
from jax import lax
from jax.experimental import pallas as pl
from jax.experimental.pallas import tpu as pltpu
from jax.experimental.pallas import tpu_sc as plsc

N = 10000
E = 320000
D = 128
NC = 2
NS = 16
NW = NC * NS
CH = 128
NCHUNK = 80
EPW = NCHUNK * CH
EPAD = NW * EPW
NPAD = 10240
RPT = NPAD // NS
RB = 512
NG = NPAD // RB

_mesh = plsc.VectorSubcoreMesh(core_axis_name="c", subcore_axis_name="s")


@functools.partial(
    pl.kernel,
    out_type=jax.ShapeDtypeStruct((NC, NPAD), jnp.float32),
    mesh=_mesh,
    compiler_params=pltpu.CompilerParams(needs_layout_passes=False),
    scratch_types=[
        pltpu.VMEM((EPW,), jnp.int32),
        pltpu.VMEM((EPW,), jnp.float32),
        pltpu.VMEM((NPAD,), jnp.float32),
        pltpu.VMEM((NS, RPT), jnp.float32),
        pltpu.VMEM((RPT,), jnp.float32),
        pltpu.VMEM_SHARED((NS, NPAD), jnp.float32),
    ],
)
def _deg_kernel(dst_f, ew_f, degp, dstv, eww, degv, redv, outv, shared):
    c = lax.axis_index("c")
    s = lax.axis_index("s")
    wid = c * NS + s
    pltpu.sync_copy(dst_f.at[wid], dstv)
    pltpu.sync_copy(ew_f.at[wid], eww)

    zero16 = jnp.zeros((16,), jnp.float32)

    def _zero(i, _):
        degv[pl.ds(i * 16, 16)] = zero16
        return 0

    lax.fori_loop(0, NPAD // 16, _zero, 0)

    def _acc(i, _):
        d16 = dstv[pl.ds(i * 16, 16)]
        w16 = eww[pl.ds(i * 16, 16)]
        plsc.addupdate_scatter(degv, [d16], w16)
        return 0

    lax.fori_loop(0, EPW // 16, _acc, 0)

    pltpu.sync_copy(degv, shared.at[s])
    plsc.subcore_barrier()

    for p in range(NS):
        pltpu.sync_copy(shared.at[p, pl.ds(s * RPT, RPT)], redv.at[p])

    def _red(j, _):
        sl = pl.ds(j * 16, 16)
        acc = redv[0, sl]
        for p in range(1, NS):
            acc = acc + redv[p, sl]
        outv[sl] = acc
        return 0

    lax.fori_loop(0, RPT // 16, _red, 0)
    pltpu.sync_copy(outv, degp.at[c, pl.ds(s * RPT, RPT)])


@functools.partial(
    pl.kernel,
    out_type=jax.ShapeDtypeStruct((NC, NPAD, D), jnp.float32),
    mesh=_mesh,
    compiler_params=pltpu.CompilerParams(needs_layout_passes=False),
    scratch_types=[
        pltpu.VMEM((NCHUNK, CH), jnp.int32),
        pltpu.VMEM((NCHUNK, CH), jnp.int32),
        pltpu.VMEM((NCHUNK, CH), jnp.float32),
        pltpu.VMEM((CH, D), jnp.float32),
        pltpu.VMEM_SHARED((NPAD, D), jnp.float32),
        pltpu.SemaphoreType.DMA,
    ],
)
def _agg(src_r, dst_r, ew_r, y, part, srcv, dstv, eww, rows, shared, sem):
    c = lax.axis_index("c")
    s = lax.axis_index("s")
    wid = c * NS + s
    pltpu.sync_copy(src_r.at[wid], srcv)
    pltpu.sync_copy(dst_r.at[wid], dstv)
    pltpu.sync_copy(ew_r.at[wid], eww)

    zero16 = jnp.zeros((16,), jnp.float32)

    def _zrow(r, _):
        for k in range(D // 16):
            rows[r, pl.ds(k * 16, 16)] = zero16
        return 0

    lax.fori_loop(0, CH, _zrow, 0)
    for t in range(RPT // CH):
        pltpu.sync_copy(rows, shared.at[pl.ds(s * RPT + t * CH, CH)])
    plsc.subcore_barrier()

    def _chunk(i, _):
        pltpu.sync_copy(y.at[srcv.at[i]], rows)

        def _scale(r, _):
            w = plsc.load_gather(eww.at[i], [jnp.full((16,), r, jnp.int32)])
            for k in range(D // 16):
                sl = pl.ds(k * 16, 16)
                rows[r, sl] = rows[r, sl] * w
            return 0

        lax.fori_loop(0, CH, _scale, 0)
        pltpu.sync_copy(rows, shared.at[dstv.at[i]], add=True)
        return 0

    lax.fori_loop(0, NCHUNK, _chunk, 0)
    plsc.subcore_barrier()
    pltpu.sync_copy(shared.at[pl.ds(s * RPT, RPT)],
                    part.at[c, pl.ds(s * RPT, RPT)])


def _tc1_body(x_ref, w_ref, d0_ref, d1_ref, y_ref):
    deg = d0_ref[...] + d1_ref[...] + 1.0
    dinv = lax.rsqrt(deg)
    y_ref[...] = jnp.dot(x_ref[...], w_ref[...],
                         preferred_element_type=jnp.float32) * dinv


def _tc2_body(p0_ref, p1_ref, y1_ref, d0_ref, d1_ref, b_ref, w_ref, y2_ref):
    deg = d0_ref[...] + d1_ref[...] + 1.0
    dinv = lax.rsqrt(deg)
    h = jnp.maximum(dinv * (p0_ref[...] + p1_ref[...] + y1_ref[...])
                    + b_ref[...], 0.0)
    y2_ref[...] = jnp.dot(h, w_ref[...],
                          preferred_element_type=jnp.float32) * dinv


def _tc3_body(q0_ref, q1_ref, y2_ref, d0_ref, d1_ref, b_ref, o_ref):
    deg = d0_ref[...] + d1_ref[...] + 1.0
    dinv = lax.rsqrt(deg)
    o_ref[...] = dinv * (q0_ref[...] + q1_ref[...] + y2_ref[...]) + b_ref[...]


_row = pl.BlockSpec((RB, D), lambda i: (i, 0))
_col = pl.BlockSpec((RB, 1), lambda i: (i, 0))
_full = pl.BlockSpec((D, D), lambda i: (0, 0))
_bias = pl.BlockSpec((1, D), lambda i: (0, 0))
_out128 = jax.ShapeDtypeStruct((NPAD, D), jnp.float32)

_tc1 = pl.pallas_call(
    _tc1_body, grid=(NG,),
    in_specs=[_row, _full, _col, _col],
    out_specs=_row, out_shape=_out128)

_tc2 = pl.pallas_call(
    _tc2_body, grid=(NG,),
    in_specs=[_row, _row, _row, _col, _col, _bias, _full],
    out_specs=_row, out_shape=_out128)

_tc3 = pl.pallas_call(
    _tc3_body, grid=(NG,),
    in_specs=[_row, _row, _row, _col, _col, _bias],
    out_specs=_row, out_shape=_out128)


@jax.jit
def kernel(x, edge_idx, edge_wt, W1, b1, W2, b2):
    pad = EPAD - E
    src = jnp.pad(edge_idx[0], (0, pad))
    dst = jnp.pad(edge_idx[1], (0, pad))
    ew = jnp.pad(edge_wt, (0, pad))
    src_r = src.reshape(NW, NCHUNK, CH)
    dst_r = dst.reshape(NW, NCHUNK, CH)
    ew_r = ew.reshape(NW, NCHUNK, CH)
    dst_f = dst.reshape(NW, EPW)
    ew_f = ew.reshape(NW, EPW)

    x_pad = jnp.pad(x, ((0, NPAD - N), (0, 0)))
    b1r = b1.reshape(1, D)
    b2r = b2.reshape(1, D)

    degp = _deg_kernel(dst_f, ew_f)
    d0 = degp[0].reshape(NPAD, 1)
    d1 = degp[1].reshape(NPAD, 1)

    y1 = _tc1(x_pad, W1, d0, d1)
    p = _agg(src_r, dst_r, ew_r, y1)
    y2 = _tc2(p[0], p[1], y1, d0, d1, b1r, W2)
    q = _agg(src_r, dst_r, ew_r, y2)
    out = _tc3(q[0], q[1], y2, d0, d1, b2r)
    return out[:N]


# speedup vs baseline: 7.9989x; 1.2197x over previous
import functools
import jax
import jax.numpy as jnp
from jax import lax
from jax.experimental import pallas as pl
from jax.experimental.pallas import tpu as pltpu
from jax.experimental.pallas import tpu_sc as plsc

N = 10000
E = 320000
D = 128
NC = 2
NS = 16
NW = NC * NS
CH = 128
NCHUNK = 80
EPW = NCHUNK * CH
EPAD = NW * EPW
NPAD = 10240
RPT = NPAD // NS
RB = 512
NG = NPAD // RB

_mesh = plsc.VectorSubcoreMesh(core_axis_name="c", subcore_axis_name="s")


@functools.partial(
    pl.kernel,
    out_type=jax.ShapeDtypeStruct((NC, NPAD), jnp.float32),
    mesh=_mesh,
    compiler_params=pltpu.CompilerParams(needs_layout_passes=False),
    scratch_types=[
        pltpu.VMEM((EPW,), jnp.int32),
        pltpu.VMEM((EPW,), jnp.float32),
        pltpu.VMEM((NPAD,), jnp.float32),
        pltpu.VMEM((NS, RPT), jnp.float32),
        pltpu.VMEM((RPT,), jnp.float32),
        pltpu.VMEM_SHARED((NS, NPAD), jnp.float32),
    ],
)
def _deg_kernel(dst_f, ew_f, degp, dstv, eww, degv, redv, outv, shared):
    c = lax.axis_index("c")
    s = lax.axis_index("s")
    wid = c * NS + s
    pltpu.sync_copy(dst_f.at[wid], dstv)
    pltpu.sync_copy(ew_f.at[wid], eww)

    zero16 = jnp.zeros((16,), jnp.float32)

    def _zero(i, _):
        degv[pl.ds(i * 16, 16)] = zero16
        return 0

    lax.fori_loop(0, NPAD // 16, _zero, 0)

    def _acc(i, _):
        d16 = dstv[pl.ds(i * 16, 16)]
        w16 = eww[pl.ds(i * 16, 16)]
        plsc.addupdate_scatter(degv, [d16], w16)
        return 0

    lax.fori_loop(0, EPW // 16, _acc, 0)

    pltpu.sync_copy(degv, shared.at[s])
    plsc.subcore_barrier()

    for p in range(NS):
        pltpu.sync_copy(shared.at[p, pl.ds(s * RPT, RPT)], redv.at[p])

    def _red(j, _):
        sl = pl.ds(j * 16, 16)
        acc = redv[0, sl]
        for p in range(1, NS):
            acc = acc + redv[p, sl]
        outv[sl] = acc
        return 0

    lax.fori_loop(0, RPT // 16, _red, 0)
    pltpu.sync_copy(outv, degp.at[c, pl.ds(s * RPT, RPT)])


PH = NCHUNK // 2


@functools.partial(
    pl.kernel,
    out_type=jax.ShapeDtypeStruct((NC, NPAD, D), jnp.float32),
    mesh=_mesh,
    compiler_params=pltpu.CompilerParams(needs_layout_passes=False),
    scratch_types=[
        pltpu.VMEM((PH, CH), jnp.int32),
        pltpu.VMEM((PH, CH), jnp.int32),
        pltpu.VMEM((PH, CH), jnp.float32),
        pltpu.VMEM((CH, D), jnp.float32),
        pltpu.VMEM((CH, D), jnp.float32),
        pltpu.VMEM_SHARED((NPAD, D), jnp.float32),
        pltpu.SemaphoreType.DMA,
        pltpu.SemaphoreType.DMA,
        pltpu.SemaphoreType.DMA,
        pltpu.SemaphoreType.DMA,
    ],
)
def _agg(src_r, dst_r, ew_r, y, part, srcv, dstv, eww, rows0, rows1,
         shared, g0, g1, s0, s1):
    c = lax.axis_index("c")
    s = lax.axis_index("s")
    wid = c * NS + s

    zero16 = jnp.zeros((16,), jnp.float32)

    def _zrow(r, _):
        for k in range(D // 16):
            rows0[r, pl.ds(k * 16, 16)] = zero16
        return 0

    lax.fori_loop(0, CH, _zrow, 0)
    for t in range(RPT // CH):
        pltpu.sync_copy(rows0, shared.at[pl.ds(s * RPT + t * CH, CH)])
    plsc.subcore_barrier()

    def _scale(rows_b, ci):
        @plsc.parallel_loop(0, CH, 1, unroll=2)
        def _row(r):
            w = plsc.load_gather(eww.at[ci], [jnp.full((16,), r, jnp.int32)])
            for k in range(D // 16):
                sl = pl.ds(k * 16, 16)
                rows_b[r, sl] = rows_b[r, sl] * w

    for p in range(2):
        base = p * PH
        pltpu.sync_copy(src_r.at[wid, pl.ds(base, PH)], srcv)
        pltpu.sync_copy(dst_r.at[wid, pl.ds(base, PH)], dstv)
        pltpu.sync_copy(ew_r.at[wid, pl.ds(base, PH)], eww)
        pltpu.async_copy(y.at[srcv.at[0]], rows0, g0)
        pltpu.async_copy(y.at[srcv.at[1]], rows1, g1)

        def _pair(j, _):
            i0 = 2 * j
            i1 = i0 + 1
            pltpu.make_async_copy(y.at[srcv.at[i0]], rows0, g0).wait()
            _scale(rows0, i0)
            pltpu.async_copy(rows0, shared.at[dstv.at[i0]], s0, add=True)
            pltpu.make_async_copy(y.at[srcv.at[i1]], rows1, g1).wait()
            _scale(rows1, i1)
            pltpu.async_copy(rows1, shared.at[dstv.at[i1]], s1, add=True)
            pltpu.make_async_copy(rows0, shared.at[dstv.at[i0]], s0).wait()

            @pl.when(i0 + 2 < PH)
            def _():
                pltpu.async_copy(y.at[srcv.at[i0 + 2]], rows0, g0)

            pltpu.make_async_copy(rows1, shared.at[dstv.at[i1]], s1).wait()

            @pl.when(i1 + 2 < PH)
            def _():
                pltpu.async_copy(y.at[srcv.at[i1 + 2]], rows1, g1)

            return 0

        lax.fori_loop(0, PH // 2, _pair, 0)

    plsc.subcore_barrier()
    pltpu.sync_copy(shared.at[pl.ds(s * RPT, RPT)],
                    part.at[c, pl.ds(s * RPT, RPT)])


def _tc1_body(x_ref, w_ref, d0_ref, d1_ref, y_ref):
    deg = d0_ref[...] + d1_ref[...] + 1.0
    dinv = lax.rsqrt(deg)
    y_ref[...] = jnp.dot(x_ref[...], w_ref[...],
                         preferred_element_type=jnp.float32) * dinv


def _tc2_body(p0_ref, p1_ref, y1_ref, d0_ref, d1_ref, b_ref, w_ref, y2_ref):
    deg = d0_ref[...] + d1_ref[...] + 1.0
    dinv = lax.rsqrt(deg)
    h = jnp.maximum(dinv * (p0_ref[...] + p1_ref[...] + y1_ref[...])
                    + b_ref[...], 0.0)
    y2_ref[...] = jnp.dot(h, w_ref[...],
                          preferred_element_type=jnp.float32) * dinv


def _tc3_body(q0_ref, q1_ref, y2_ref, d0_ref, d1_ref, b_ref, o_ref):
    deg = d0_ref[...] + d1_ref[...] + 1.0
    dinv = lax.rsqrt(deg)
    o_ref[...] = dinv * (q0_ref[...] + q1_ref[...] + y2_ref[...]) + b_ref[...]


_row = pl.BlockSpec((RB, D), lambda i: (i, 0))
_col = pl.BlockSpec((RB, 1), lambda i: (i, 0))
_full = pl.BlockSpec((D, D), lambda i: (0, 0))
_bias = pl.BlockSpec((1, D), lambda i: (0, 0))
_out128 = jax.ShapeDtypeStruct((NPAD, D), jnp.float32)

_tc1 = pl.pallas_call(
    _tc1_body, grid=(NG,),
    in_specs=[_row, _full, _col, _col],
    out_specs=_row, out_shape=_out128)

_tc2 = pl.pallas_call(
    _tc2_body, grid=(NG,),
    in_specs=[_row, _row, _row, _col, _col, _bias, _full],
    out_specs=_row, out_shape=_out128)

_tc3 = pl.pallas_call(
    _tc3_body, grid=(NG,),
    in_specs=[_row, _row, _row, _col, _col, _bias],
    out_specs=_row, out_shape=_out128)


@jax.jit
def kernel(x, edge_idx, edge_wt, W1, b1, W2, b2):
    pad = EPAD - E
    src = jnp.pad(edge_idx[0], (0, pad))
    dst = jnp.pad(edge_idx[1], (0, pad))
    ew = jnp.pad(edge_wt, (0, pad))
    src_r = src.reshape(NW, NCHUNK, CH)
    dst_r = dst.reshape(NW, NCHUNK, CH)
    ew_r = ew.reshape(NW, NCHUNK, CH)
    dst_f = dst.reshape(NW, EPW)
    ew_f = ew.reshape(NW, EPW)

    x_pad = jnp.pad(x, ((0, NPAD - N), (0, 0)))
    b1r = b1.reshape(1, D)
    b2r = b2.reshape(1, D)

    degp = _deg_kernel(dst_f, ew_f)
    d0 = degp[0].reshape(NPAD, 1)
    d1 = degp[1].reshape(NPAD, 1)

    y1 = _tc1(x_pad, W1, d0, d1)
    p = _agg(src_r, dst_r, ew_r, y1)
    y2 = _tc2(p[0], p[1], y1, d0, d1, b1r, W2)
    q = _agg(src_r, dst_r, ew_r, y2)
    out = _tc3(q[0], q[1], y2, d0, d1, b2r)
    return out[:N]
